# baseline (speedup 1.0000x reference)
import jax
import jax.numpy as jnp
from jax import lax
from jax.experimental import pallas as pl
from jax.experimental.pallas import tpu as pltpu
from jax.experimental.pallas import tpu_sc as plsc

N_TOPICS = 8
D_IN = 1024
D_HID = 1024
D_OUT = 3
N_TOK = 8192
DP = 128

NC = 2
NS = 16
L = 16
NW = NC * NS
CHUNK = N_TOK // NW
NGR = CHUNK // L
BATCHES = (48, 48, 48, 48, 32, 32)
BOFF = (0, 48, 96, 144, 192, 224)
BMAX = max(BATCHES)

T = 256
NWORK = N_TOK // T + (N_TOPICS - 1) + 1
NPAD = NWORK * T
EIDN = 48


GT = 1024
GSUB = GT // CHUNK
DH = D_IN // 2


def _gate_body(x_ref, wsel_ref, top_ref, hist_ref, xpk_ref):
    xv = x_ref[...]
    xu = lax.bitcast_convert_type(xv, jnp.uint32)
    rnd = (xu + 0x7FFF + ((xu >> 16) & 1)) >> 16
    pk = rnd[:, :DH] | (rnd[:, DH:] << 16)
    xpk_ref[...] = lax.bitcast_convert_type(pk, jnp.float32)
    logits = jnp.dot(xv, wsel_ref[...], preferred_element_type=jnp.float32)
    n = logits.shape[0]
    ids8 = lax.broadcasted_iota(jnp.int32, (n, N_TOPICS), 1)
    best = jnp.max(logits, axis=1, keepdims=True)
    idx = jnp.min(jnp.where(logits == best, ids8, N_TOPICS),
                  axis=1, keepdims=True)
    top_ref[...] = idx
    ids = lax.broadcasted_iota(jnp.int32, (n, L), 1)
    onehot = jnp.where(jnp.broadcast_to(idx, ids.shape) == ids, 1, 0)
    for s in range(GSUB):
        hist_ref[0, s:s + 1, :] = jnp.sum(
            onehot[s * CHUNK:(s + 1) * CHUNK], axis=0, keepdims=True)


def _gating(x, W_sel):
    out, hist, xpk = pl.pallas_call(
        _gate_body,
        grid=(N_TOK // GT,),
        in_specs=[
            pl.BlockSpec((GT, D_IN), lambda t: (t, 0)),
            pl.BlockSpec((D_IN, N_TOPICS), lambda t: (0, 0)),
        ],
        out_specs=[
            pl.BlockSpec((GT, 1), lambda t: (t, 0)),
            pl.BlockSpec((1, GSUB, L), lambda t: (t, 0, 0)),
            pl.BlockSpec((GT, DH), lambda t: (t, 0)),
        ],
        out_shape=[
            jax.ShapeDtypeStruct((N_TOK, 1), jnp.int32),
            jax.ShapeDtypeStruct((N_TOK // GT, GSUB, L), jnp.int32),
            jax.ShapeDtypeStruct((N_TOK, DH), jnp.float32),
        ],
    )(x, W_sel)
    return out.reshape(N_TOK), hist.reshape(NW * L), xpk


def _route_body(topics_hbm, hist_hbm, x_hbm, dst_hbm, eid_hbm, xpad_hbm,
                tv, hv, dv, i0, i1, i2, i3, i4, i5, eidv,
                xbuf0, xbuf1, lsem, ssem):
    wid = lax.axis_index("s") * NC + lax.axis_index("c")
    base = wid * CHUNK
    lane = lax.iota(jnp.int32, L)
    idx_refs = [i0, i1, i2, i3, i4, i5]
    xbufs = [xbuf0, xbuf1]

    pltpu.sync_copy(topics_hbm.at[pl.ds(base, CHUNK)], tv)
    pltpu.sync_copy(hist_hbm, hv)

    widv = jnp.full((L,), wid, jnp.int32)
    totals = jnp.zeros((L,), jnp.int32)
    before = jnp.zeros((L,), jnp.int32)
    for w in range(NW):
        row = hv[pl.ds(w * L, L)]
        totals = totals + row
        wv = jnp.full((L,), w, jnp.int32)
        before = before + jnp.where(wv < widv, row, 0)
    pc = ((totals + (T - 1)) // T) * T
    pad_off = plsc.cumsum(pc) - pc
    startv = pad_off + before

    tile_end = plsc.cumsum(pc // T)
    for grp in range(EIDN // L):
        j = lane + grp * L
        acc = jnp.zeros((L,), jnp.int32)
        for e in range(N_TOPICS):
            te = jnp.sum(jnp.where(lane == e, tile_end, 0))
            acc = acc + jnp.where(j >= te, 1, 0)
        eidv[pl.ds(grp * L, L)] = jnp.minimum(acc, N_TOPICS - 1)

    @pl.when(wid == 0)
    def _store_eid():
        pltpu.sync_copy(eidv, eid_hbm)

    for g in range(NGR):
        t16 = tv[pl.ds(g * L, L)]
        d16 = jnp.zeros((L,), jnp.int32)
        for e in range(N_TOPICS):
            m = t16 == e
            mi = jnp.where(m, 1, 0)
            csum = plsc.cumsum(mi)
            base_e = jnp.sum(jnp.where(lane == e, startv, 0))
            d16 = jnp.where(m, base_e + csum - 1, d16)
            startv = startv + jnp.where(lane == e, jnp.sum(mi), 0)
        d16 = jnp.clip(d16, 0, NPAD - 1)
        dv[pl.ds(g * L, L)] = d16
        row = g * L
        b = max(i for i in range(len(BOFF)) if BOFF[i] <= row)
        idx_refs[b][pl.ds(row - BOFF[b], L)] = d16

    pltpu.sync_copy(dv, dst_hbm.at[pl.ds(base, CHUNK)])

    nb = len(BATCHES)

    def _load(r):
        sz = BATCHES[r]
        return pltpu.async_copy(
            x_hbm.at[pl.ds(base + BOFF[r], sz)],
            xbufs[r % 2].at[pl.ds(0, sz)], lsem)

    loads = [_load(0)]
    scats = []
    for r in range(nb):
        if r + 1 < nb:
            if r >= 1:
                scats[r - 1].wait()
            loads.append(_load(r + 1))
        loads[r].wait()
        scats.append(pltpu.async_copy(
            xbufs[r % 2].at[pl.ds(0, BATCHES[r])],
            xpad_hbm.at[idx_refs[r]], ssem))
    scats[nb - 2].wait()
    scats[nb - 1].wait()


def _route(topics, hist, x):
    mesh = plsc.VectorSubcoreMesh(core_axis_name="c", subcore_axis_name="s")
    fn = pl.kernel(
        _route_body,
        out_type=[
            jax.ShapeDtypeStruct((N_TOK,), jnp.int32),
            jax.ShapeDtypeStruct((EIDN,), jnp.int32),
            jax.ShapeDtypeStruct((NPAD, DH), jnp.float32),
        ],
        mesh=mesh,
        scratch_types=(
            [
                pltpu.VMEM((CHUNK,), jnp.int32),
                pltpu.VMEM((NW * L,), jnp.int32),
                pltpu.VMEM((CHUNK,), jnp.int32),
            ]
            + [pltpu.VMEM((sz,), jnp.int32) for sz in BATCHES]
            + [
                pltpu.VMEM((EIDN,), jnp.int32),
                pltpu.VMEM((BMAX, DH), jnp.float32),
                pltpu.VMEM((BMAX, DH), jnp.float32),
                pltpu.SemaphoreType.DMA,
                pltpu.SemaphoreType.DMA,
            ]
        ),
        compiler_params=pltpu.CompilerParams(needs_layout_passes=False),
    )
    return fn(topics, hist, x)


DO8 = 8


def _mlp_body(eid_ref, x_ref, w1_ref, b1_ref, w2_ref, b2_ref, out_ref, w1b):
    w = pl.program_id(0)
    cur = eid_ref[w]
    prev = eid_ref[jnp.maximum(w - 1, 0)]

    @pl.when((w == 0) | (cur != prev))
    def _cast_w1():
        w1b[...] = w1_ref[0].astype(jnp.bfloat16)

    xu = lax.bitcast_convert_type(x_ref[...], jnp.uint32)
    xlo = lax.bitcast_convert_type(xu << 16, jnp.float32) \
        .astype(jnp.bfloat16)
    xhi = lax.bitcast_convert_type(xu & jnp.uint32(0xFFFF0000),
                                   jnp.float32).astype(jnp.bfloat16)
    h = jnp.maximum(
        jnp.dot(xlo, w1b[0:DH, :], preferred_element_type=jnp.float32)
        + jnp.dot(xhi, w1b[DH:D_IN, :], preferred_element_type=jnp.float32)
        + b1_ref[0], 0.0)
    o = jnp.dot(h, w2_ref[0], preferred_element_type=jnp.float32) + b2_ref[0]
    out_ref[:, 0:DO8] = o


def _mlp(eid, x_pad, W1, b1r, W2p, b2p):
    grid_spec = pltpu.PrefetchScalarGridSpec(
        num_scalar_prefetch=1,
        grid=(NWORK,),
        in_specs=[
            pl.BlockSpec((T, DH), lambda w, eid: (w, 0)),
            pl.BlockSpec((1, D_IN, D_HID), lambda w, eid: (eid[w], 0, 0)),
            pl.BlockSpec((1, 1, D_HID), lambda w, eid: (eid[w], 0, 0)),
            pl.BlockSpec((1, D_HID, DO8), lambda w, eid: (eid[w], 0, 0)),
            pl.BlockSpec((1, 1, DO8), lambda w, eid: (eid[w], 0, 0)),
        ],
        out_specs=pl.BlockSpec((T, DP), lambda w, eid: (w, 0)),
        scratch_shapes=[pltpu.VMEM((D_IN, D_HID), jnp.bfloat16)],
    )
    return pl.pallas_call(
        _mlp_body,
        grid_spec=grid_spec,
        out_shape=jax.ShapeDtypeStruct((NPAD, DP), jnp.float32),
    )(eid, x_pad, W1, b1r, W2p, b2p)


def _ungather_body(dst_hbm, pp_hbm, out_hbm, idxv, buf, sem):
    wid = lax.axis_index("s") * NC + lax.axis_index("c")
    base = wid * CHUNK
    pltpu.sync_copy(dst_hbm.at[pl.ds(base, CHUNK)], idxv)
    pltpu.async_copy(pp_hbm.at[idxv], buf, sem).wait()
    pltpu.sync_copy(buf, out_hbm.at[pl.ds(base, CHUNK)])


def _ungather(dst, preds_pad):
    mesh = plsc.VectorSubcoreMesh(core_axis_name="c", subcore_axis_name="s")
    fn = pl.kernel(
        _ungather_body,
        out_type=jax.ShapeDtypeStruct((N_TOK, DP), jnp.float32),
        mesh=mesh,
        scratch_types=[
            pltpu.VMEM((CHUNK,), jnp.int32),
            pltpu.VMEM((CHUNK, DP), jnp.float32),
            pltpu.SemaphoreType.DMA,
        ],
        compiler_params=pltpu.CompilerParams(needs_layout_passes=False),
    )
    return fn(dst, preds_pad)


def kernel(x, W_sel, W1, b1, W2, b2):
    topics, hist, xpk = _gating(x, W_sel)
    dst, eid, x_pad = _route(topics, hist, xpk)
    b1r = b1.reshape(N_TOPICS, 1, D_HID)
    W2b = jnp.pad(W2, ((0, 0), (0, 0), (0, DO8 - D_OUT)))
    b2p = jnp.pad(b2, ((0, 0), (0, DO8 - D_OUT))).reshape(N_TOPICS, 1, DO8)
    preds_pad = _mlp(eid[:NWORK], x_pad, W1, b1r, W2b, b2p)
    out16 = _ungather(dst, preds_pad)
    return out16[:, :D_OUT]


# speedup vs baseline: 1.0442x; 1.0442x over previous
import jax
import jax.numpy as jnp
from jax import lax
from jax.experimental import pallas as pl
from jax.experimental.pallas import tpu as pltpu
from jax.experimental.pallas import tpu_sc as plsc

N_TOPICS = 8
D_IN = 1024
D_HID = 1024
D_OUT = 3
N_TOK = 8192
DP = 128

NC = 2
NS = 16
L = 16
NW = NC * NS
CHUNK = N_TOK // NW
NGR = CHUNK // L
BATCHES = (48, 48, 48, 48, 32, 32)
BOFF = (0, 48, 96, 144, 192, 224)
BMAX = max(BATCHES)

T = 256
NWORK = N_TOK // T + (N_TOPICS - 1)
NPAD = NWORK * T
EIDN = 48


GT = 1024
GSUB = GT // CHUNK
DH = D_IN // 2


def _gate_body(x_ref, wsel_ref, top_ref, hist_ref, xpk_ref):
    xv = x_ref[...]
    xu = lax.bitcast_convert_type(xv, jnp.uint32)
    rnd = (xu + 0x7FFF + ((xu >> 16) & 1)) >> 16
    pk = rnd[:, :DH] | (rnd[:, DH:] << 16)
    xpk_ref[...] = lax.bitcast_convert_type(pk, jnp.float32)
    logits = jnp.dot(xv, wsel_ref[...], preferred_element_type=jnp.float32)
    n = logits.shape[0]
    ids8 = lax.broadcasted_iota(jnp.int32, (n, N_TOPICS), 1)
    best = jnp.max(logits, axis=1, keepdims=True)
    idx = jnp.min(jnp.where(logits == best, ids8, N_TOPICS),
                  axis=1, keepdims=True)
    top_ref[...] = idx
    ids = lax.broadcasted_iota(jnp.int32, (n, L), 1)
    onehot = jnp.where(jnp.broadcast_to(idx, ids.shape) == ids, 1, 0)
    for s in range(GSUB):
        hist_ref[0, s:s + 1, :] = jnp.sum(
            onehot[s * CHUNK:(s + 1) * CHUNK], axis=0, keepdims=True)


def _gating(x, W_sel):
    out, hist, xpk = pl.pallas_call(
        _gate_body,
        grid=(N_TOK // GT,),
        in_specs=[
            pl.BlockSpec((GT, D_IN), lambda t: (t, 0)),
            pl.BlockSpec((D_IN, N_TOPICS), lambda t: (0, 0)),
        ],
        out_specs=[
            pl.BlockSpec((GT, 1), lambda t: (t, 0)),
            pl.BlockSpec((1, GSUB, L), lambda t: (t, 0, 0)),
            pl.BlockSpec((GT, DH), lambda t: (t, 0)),
        ],
        out_shape=[
            jax.ShapeDtypeStruct((N_TOK, 1), jnp.int32),
            jax.ShapeDtypeStruct((N_TOK // GT, GSUB, L), jnp.int32),
            jax.ShapeDtypeStruct((N_TOK, DH), jnp.float32),
        ],
    )(x, W_sel)
    return out.reshape(N_TOK), hist.reshape(NW * L), xpk


def _route_body(topics_hbm, hist_hbm, x_hbm, dst_hbm, eid_hbm, xpad_hbm,
                tv, hv, dv, i0, i1, i2, i3, i4, i5, eidv,
                xbuf0, xbuf1, lsem, ssem):
    wid = lax.axis_index("s") * NC + lax.axis_index("c")
    base = wid * CHUNK
    lane = lax.iota(jnp.int32, L)
    idx_refs = [i0, i1, i2, i3, i4, i5]
    xbufs = [xbuf0, xbuf1]

    pltpu.sync_copy(topics_hbm.at[pl.ds(base, CHUNK)], tv)
    pltpu.sync_copy(hist_hbm, hv)

    widv = jnp.full((L,), wid, jnp.int32)
    totals = jnp.zeros((L,), jnp.int32)
    before = jnp.zeros((L,), jnp.int32)
    for w in range(NW):
        row = hv[pl.ds(w * L, L)]
        totals = totals + row
        wv = jnp.full((L,), w, jnp.int32)
        before = before + jnp.where(wv < widv, row, 0)
    pc = ((totals + (T - 1)) // T) * T
    pad_off = plsc.cumsum(pc) - pc
    startv = pad_off + before

    tile_end = plsc.cumsum(pc // T)
    for grp in range(EIDN // L):
        j = lane + grp * L
        acc = jnp.zeros((L,), jnp.int32)
        for e in range(N_TOPICS):
            te = jnp.sum(jnp.where(lane == e, tile_end, 0))
            acc = acc + jnp.where(j >= te, 1, 0)
        eidv[pl.ds(grp * L, L)] = jnp.minimum(acc, N_TOPICS - 1)

    @pl.when(wid == 0)
    def _store_eid():
        pltpu.sync_copy(eidv, eid_hbm)

    for g in range(NGR):
        t16 = tv[pl.ds(g * L, L)]
        d16 = jnp.zeros((L,), jnp.int32)
        for e in range(N_TOPICS):
            m = t16 == e
            mi = jnp.where(m, 1, 0)
            csum = plsc.cumsum(mi)
            base_e = jnp.sum(jnp.where(lane == e, startv, 0))
            d16 = jnp.where(m, base_e + csum - 1, d16)
            startv = startv + jnp.where(lane == e, jnp.sum(mi), 0)
        d16 = jnp.clip(d16, 0, NPAD - 1)
        dv[pl.ds(g * L, L)] = d16
        row = g * L
        b = max(i for i in range(len(BOFF)) if BOFF[i] <= row)
        idx_refs[b][pl.ds(row - BOFF[b], L)] = d16

    pltpu.sync_copy(dv, dst_hbm.at[pl.ds(base, CHUNK)])

    nb = len(BATCHES)

    def _load(r):
        sz = BATCHES[r]
        return pltpu.async_copy(
            x_hbm.at[pl.ds(base + BOFF[r], sz)],
            xbufs[r % 2].at[pl.ds(0, sz)], lsem)

    loads = [_load(0)]
    scats = []
    for r in range(nb):
        if r + 1 < nb:
            if r >= 1:
                scats[r - 1].wait()
            loads.append(_load(r + 1))
        loads[r].wait()
        scats.append(pltpu.async_copy(
            xbufs[r % 2].at[pl.ds(0, BATCHES[r])],
            xpad_hbm.at[idx_refs[r]], ssem))
    scats[nb - 2].wait()
    scats[nb - 1].wait()


def _route(topics, hist, x):
    mesh = plsc.VectorSubcoreMesh(core_axis_name="c", subcore_axis_name="s")
    fn = pl.kernel(
        _route_body,
        out_type=[
            jax.ShapeDtypeStruct((N_TOK,), jnp.int32),
            jax.ShapeDtypeStruct((EIDN,), jnp.int32),
            jax.ShapeDtypeStruct((NPAD, DH), jnp.float32),
        ],
        mesh=mesh,
        scratch_types=(
            [
                pltpu.VMEM((CHUNK,), jnp.int32),
                pltpu.VMEM((NW * L,), jnp.int32),
                pltpu.VMEM((CHUNK,), jnp.int32),
            ]
            + [pltpu.VMEM((sz,), jnp.int32) for sz in BATCHES]
            + [
                pltpu.VMEM((EIDN,), jnp.int32),
                pltpu.VMEM((BMAX, DH), jnp.float32),
                pltpu.VMEM((BMAX, DH), jnp.float32),
                pltpu.SemaphoreType.DMA,
                pltpu.SemaphoreType.DMA,
            ]
        ),
        compiler_params=pltpu.CompilerParams(needs_layout_passes=False),
    )
    return fn(topics, hist, x)


DO8 = 8


def _mlp_body(eid_ref, x_ref, w1_ref, b1_ref, w2_ref, b2_ref, out_ref, w1b):
    w = pl.program_id(0)
    cur = eid_ref[w]
    prev = eid_ref[jnp.maximum(w - 1, 0)]

    @pl.when((w == 0) | (cur != prev))
    def _cast_w1():
        w1b[...] = w1_ref[0].astype(jnp.bfloat16)

    xu = lax.bitcast_convert_type(x_ref[...], jnp.uint32)
    xlo = lax.bitcast_convert_type(xu << 16, jnp.float32) \
        .astype(jnp.bfloat16)
    xhi = lax.bitcast_convert_type(xu & jnp.uint32(0xFFFF0000),
                                   jnp.float32).astype(jnp.bfloat16)
    h = jnp.maximum(
        jnp.dot(xlo, w1b[0:DH, :], preferred_element_type=jnp.float32)
        + jnp.dot(xhi, w1b[DH:D_IN, :], preferred_element_type=jnp.float32)
        + b1_ref[0], 0.0)
    o = jnp.dot(h.astype(jnp.bfloat16), w2_ref[0],
                preferred_element_type=jnp.float32) + b2_ref[0]
    out_ref[:, 0:DO8] = o


def _mlp(eid, x_pad, W1, b1r, W2p, b2p):
    grid_spec = pltpu.PrefetchScalarGridSpec(
        num_scalar_prefetch=1,
        grid=(NWORK,),
        in_specs=[
            pl.BlockSpec((T, DH), lambda w, eid: (w, 0)),
            pl.BlockSpec((1, D_IN, D_HID), lambda w, eid: (eid[w], 0, 0)),
            pl.BlockSpec((1, 1, D_HID), lambda w, eid: (eid[w], 0, 0)),
            pl.BlockSpec((1, D_HID, DO8), lambda w, eid: (eid[w], 0, 0)),
            pl.BlockSpec((1, 1, DO8), lambda w, eid: (eid[w], 0, 0)),
        ],
        out_specs=pl.BlockSpec((T, DP), lambda w, eid: (w, 0)),
        scratch_shapes=[pltpu.VMEM((D_IN, D_HID), jnp.bfloat16)],
    )
    return pl.pallas_call(
        _mlp_body,
        grid_spec=grid_spec,
        out_shape=jax.ShapeDtypeStruct((NPAD, DP), jnp.float32),
    )(eid, x_pad, W1, b1r, W2p, b2p)


def _ungather_body(dst_hbm, pp_hbm, out_hbm, idxv, buf, sem):
    wid = lax.axis_index("s") * NC + lax.axis_index("c")
    base = wid * CHUNK
    pltpu.sync_copy(dst_hbm.at[pl.ds(base, CHUNK)], idxv)
    pltpu.async_copy(pp_hbm.at[idxv], buf, sem).wait()
    pltpu.sync_copy(buf, out_hbm.at[pl.ds(base, CHUNK)])


def _ungather(dst, preds_pad):
    mesh = plsc.VectorSubcoreMesh(core_axis_name="c", subcore_axis_name="s")
    fn = pl.kernel(
        _ungather_body,
        out_type=jax.ShapeDtypeStruct((N_TOK, DP), jnp.float32),
        mesh=mesh,
        scratch_types=[
            pltpu.VMEM((CHUNK,), jnp.int32),
            pltpu.VMEM((CHUNK, DP), jnp.float32),
            pltpu.SemaphoreType.DMA,
        ],
        compiler_params=pltpu.CompilerParams(needs_layout_passes=False),
    )
    return fn(dst, preds_pad)


def kernel(x, W_sel, W1, b1, W2, b2):
    topics, hist, xpk = _gating(x, W_sel)
    dst, eid, x_pad = _route(topics, hist, xpk)
    b1r = b1.reshape(N_TOPICS, 1, D_HID)
    W2b = jnp.pad(W2, ((0, 0), (0, 0), (0, DO8 - D_OUT))).astype(jnp.bfloat16)
    b2p = jnp.pad(b2, ((0, 0), (0, DO8 - D_OUT))).reshape(N_TOPICS, 1, DO8)
    preds_pad = _mlp(eid[:NWORK], x_pad, W1, b1r, W2b, b2p)
    out16 = _ungather(dst, preds_pad)
    return out16[:, :D_OUT]
